# baseline (speedup 1.0000x reference)
import functools

import jax
import jax.numpy as jnp
from jax import lax
from jax.experimental import pallas as pl
from jax.experimental.pallas import tpu as pltpu
from jax.experimental.pallas import tpu_sc as plsc

_T = 0.1
_K = 8
_B = 128
_N = 32768
_ROWS_PER_BLOCK = 8

_NC = 2
_NS = 16
_L = 16
_NW = _NC * _NS
_RPW = _B // _NW


def _sc_mesh():
    return plsc.VectorSubcoreMesh(core_axis_name="c", subcore_axis_name="s")


def _splat_max(v, buf):
    for s in (8, 4, 2, 1):
        buf[pl.ds(0, _L)] = v
        buf[pl.ds(_L, _L)] = v
        v = jnp.maximum(v, buf[pl.ds(s, _L)])
    return v


def _sc_body(lg_hbm, d_hbm, row_v, out_v, buf_v):
    neginf = jnp.float32(-jnp.inf)
    wid = lax.axis_index("s") * _NC + lax.axis_index("c")
    for i in range(_RPW):
        row = wid * _RPW + i
        pltpu.sync_copy(lg_hbm.at[row], row_v)

        def ibody(j, nets):
            base = j * (4 * _L)
            n = list(nets)
            for t in range(4):
                x = row_v[pl.ds(base + t * _L, _L)]
                g = (t % 2) * _K
                for lvl in range(_K):
                    cur = n[g + lvl]
                    hi = jnp.maximum(cur, x)
                    x = jnp.minimum(cur, x)
                    n[g + lvl] = hi
            return tuple(n)

        init = tuple(jnp.full((_L,), neginf, jnp.float32) for _ in range(2 * _K))
        nets = lax.fori_loop(0, _N // (4 * _L), ibody, init)

        cand = list(nets)
        total = jnp.zeros((_L,), jnp.int32)
        thr = jnp.full((_L,), neginf, jnp.float32)
        for _ in range(_K):
            m = cand[0]
            for v in cand[1:]:
                m = jnp.maximum(m, v)
            mval = _splat_max(m, buf_v)
            cnt = jnp.zeros((_L,), jnp.int32)
            for v in cand:
                cnt = cnt + plsc.all_reduce_population_count(v == mval)
            done = total >= _K
            thr = jnp.where(done, thr, mval)
            total = total + jnp.where(done, jnp.zeros((_L,), jnp.int32), cnt)
            cand = [jnp.where(v == mval, neginf, v) for v in cand]

        def obody(j, carry):
            base = j * (4 * _L)
            for t in range(4):
                x = row_v[pl.ds(base + t * _L, _L)]
                out_v[pl.ds(base + t * _L, _L)] = jnp.where(
                    x >= thr, jnp.float32(1.0), jnp.float32(0.0))
            return carry

        lax.fori_loop(0, _N // (4 * _L), obody, jnp.int32(0))
        pltpu.sync_copy(out_v, d_hbm.at[row])


def _c_body(logits_ref, u_ref, c_ref):
    tiny = jnp.finfo(jnp.float32).tiny
    lg = logits_ref[...]
    u = u_ref[...]
    z = -jnp.log(-jnp.log(jnp.clip(u, tiny, 1.0 - tiny)))
    w = lg + z
    t = w / _T
    m = jnp.max(t, axis=-1, keepdims=True)
    a = jnp.exp(t - m)
    csum = jnp.zeros_like(w)
    rs = jnp.ones((a.shape[0], 1), jnp.float32)
    for k in range(_K):
        if k > 0:
            q = jnp.maximum(1.0 - p, tiny)
            q2 = q * q
            q4 = q2 * q2
            a = (a * (q4 * q4)) * (q2 * rs)
        s = jnp.maximum(jnp.sum(a, axis=-1, keepdims=True), tiny)
        rs = 1.0 / s
        p = a * rs
        csum = csum + p
    c_ref[...] = csum


def _body(logits_ref, u_ref, d_ref, c_ref):
    tiny = jnp.finfo(jnp.float32).tiny
    lg = logits_ref[...]
    u = u_ref[...]
    z = -jnp.log(-jnp.log(jnp.clip(u, tiny, 1.0 - tiny)))
    w = lg + z
    t = w / _T
    m = jnp.max(t, axis=-1, keepdims=True)
    a = jnp.exp(t - m)
    csum = jnp.zeros_like(w)
    rs = jnp.ones((a.shape[0], 1), jnp.float32)
    for k in range(_K):
        if k > 0:
            q = jnp.maximum(1.0 - p, tiny)
            q2 = q * q
            q4 = q2 * q2
            a = (a * (q4 * q4)) * (q2 * rs)
        s = jnp.maximum(jnp.sum(a, axis=-1, keepdims=True), tiny)
        rs = 1.0 / s
        p = a * rs
        csum = csum + p
    c_ref[...] = csum

    neginf = jnp.float32(-jnp.inf)
    rows = lg.shape[0]
    _NETS = 4
    _LANES = 128
    nets = [[jnp.full((rows, _LANES), neginf, jnp.float32) for _ in range(_K)]
            for _ in range(_NETS)]
    for c in range(_N // _LANES):
        x = lg[:, c * _LANES:(c + 1) * _LANES]
        net = nets[c % _NETS]
        for j in range(_K):
            hi = jnp.maximum(net[j], x)
            x = jnp.minimum(net[j], x)
            net[j] = hi
    cand = jnp.concatenate([t_ for net in nets for t_ in net], axis=1)
    total = jnp.zeros((rows, 1), jnp.int32)
    thr = jnp.full((rows, 1), neginf, jnp.float32)
    for _ in range(_K):
        m = jnp.max(cand, axis=-1, keepdims=True)
        eq = cand == m
        cnt = jnp.sum(eq.astype(jnp.int32), axis=-1, keepdims=True)
        done = total >= _K
        thr = jnp.where(done, thr, m)
        total = total + jnp.where(done, 0, cnt)
        cand = jnp.where(eq, neginf, cand)
    d_ref[...] = (lg >= thr).astype(jnp.float32)


def kernel(logits, u):
    lg = logits.reshape(_B, _N)
    uu = u.reshape(_B, _N)
    sc_dsamples = functools.partial(
        pl.kernel,
        mesh=_sc_mesh(),
        out_type=jax.ShapeDtypeStruct((_B, _N), jnp.float32),
        scratch_types=[
            pltpu.VMEM((_N,), jnp.float32),
            pltpu.VMEM((_N,), jnp.float32),
            pltpu.VMEM((2 * _L,), jnp.float32),
        ],
        compiler_params=pltpu.CompilerParams(needs_layout_passes=False),
    )(_sc_body)
    d = sc_dsamples(lg)
    grid = (_B // _ROWS_PER_BLOCK,)
    spec = pl.BlockSpec((_ROWS_PER_BLOCK, _N), lambda i: (i, 0))
    c = pl.pallas_call(
        _c_body,
        grid=grid,
        in_specs=[spec, spec],
        out_specs=spec,
        out_shape=jax.ShapeDtypeStruct((_B, _N), jnp.float32),
    )(lg, uu)
    return (d, c)


# speedup vs baseline: 1.6224x; 1.6224x over previous
import functools

import jax
import jax.numpy as jnp
from jax import lax
from jax.experimental import pallas as pl
from jax.experimental.pallas import tpu as pltpu
from jax.experimental.pallas import tpu_sc as plsc

_T = 0.1
_K = 8
_B = 128
_N = 32768
_ROWS_PER_BLOCK = 8

_NC = 2
_NS = 16
_L = 16
_NW = _NC * _NS
_RPW = _B // _NW


def _sc_mesh():
    return plsc.VectorSubcoreMesh(core_axis_name="c", subcore_axis_name="s")


def _splat_max(v, buf):
    for s in (8, 4, 2, 1):
        buf[pl.ds(0, _L)] = v
        buf[pl.ds(_L, _L)] = v
        v = jnp.maximum(v, buf[pl.ds(s, _L)])
    return v


def _sc_body(lg_hbm, d_hbm, row_v, out_v, buf_v):
    neginf = jnp.float32(-jnp.inf)
    wid = lax.axis_index("s") * _NC + lax.axis_index("c")
    for i in range(_RPW):
        row = wid * _RPW + i
        pltpu.sync_copy(lg_hbm.at[row], row_v)

        def ibody(j, nets):
            n = list(nets)
            for t in range(8):
                x = row_v[j, pl.ds(t * _L, _L)]
                g = (t % 2) * _K
                for lvl in range(_K):
                    cur = n[g + lvl]
                    hi = jnp.maximum(cur, x)
                    x = jnp.minimum(cur, x)
                    n[g + lvl] = hi
            return tuple(n)

        init = tuple(jnp.full((_L,), neginf, jnp.float32) for _ in range(2 * _K))
        nets = lax.fori_loop(0, _N // (8 * _L), ibody, init)

        cand = list(nets)
        total = jnp.zeros((_L,), jnp.int32)
        thr = jnp.full((_L,), neginf, jnp.float32)
        for _ in range(_K):
            m = cand[0]
            for v in cand[1:]:
                m = jnp.maximum(m, v)
            mval = _splat_max(m, buf_v)
            cnt = jnp.zeros((_L,), jnp.int32)
            for v in cand:
                cnt = cnt + plsc.all_reduce_population_count(v == mval)
            done = total >= _K
            thr = jnp.where(done, thr, mval)
            total = total + jnp.where(done, jnp.zeros((_L,), jnp.int32), cnt)
            cand = [jnp.where(v == mval, neginf, v) for v in cand]

        def obody(j, carry):
            for t in range(8):
                x = row_v[j, pl.ds(t * _L, _L)]
                out_v[pl.ds(j * 8 * _L + t * _L, _L)] = jnp.where(
                    x >= thr, jnp.float32(1.0), jnp.float32(0.0))
            return carry

        lax.fori_loop(0, _N // (8 * _L), obody, jnp.int32(0))
        pltpu.sync_copy(out_v, d_hbm.at[row])


def _c_body(logits_ref, u_ref, c_ref):
    tiny = jnp.finfo(jnp.float32).tiny
    lg = logits_ref[...]
    u = u_ref[...]
    z = -jnp.log(-jnp.log(jnp.clip(u, tiny, 1.0 - tiny)))
    w = lg + z
    t = w / _T
    m = jnp.max(jnp.max(t, axis=2, keepdims=True), axis=1, keepdims=True)
    a = jnp.exp(t - m)
    csum = jnp.zeros_like(w)
    rs = jnp.ones((a.shape[0], 1, 1), jnp.float32)
    for k in range(_K):
        if k > 0:
            q = jnp.maximum(1.0 - p, tiny)
            q2 = q * q
            q4 = q2 * q2
            a = (a * (q4 * q4)) * (q2 * rs)
        s = jnp.maximum(
            jnp.sum(jnp.sum(a, axis=2, keepdims=True), axis=1, keepdims=True),
            tiny)
        rs = 1.0 / s
        p = a * rs
        csum = csum + p
    c_ref[...] = csum.reshape(csum.shape[0], _N)


def kernel(logits, u):
    lg3 = logits.reshape(_B, _N // 128, 128)
    uu3 = u.reshape(_B, _N // 128, 128)
    sc_dsamples = functools.partial(
        pl.kernel,
        mesh=_sc_mesh(),
        out_type=jax.ShapeDtypeStruct((_B, _N), jnp.float32),
        scratch_types=[
            pltpu.VMEM((_N // 128, 128), jnp.float32),
            pltpu.VMEM((_N,), jnp.float32),
            pltpu.VMEM((2 * _L,), jnp.float32),
        ],
        compiler_params=pltpu.CompilerParams(needs_layout_passes=False),
    )(_sc_body)
    d = sc_dsamples(lg3)
    grid = (_B // _ROWS_PER_BLOCK,)
    spec = pl.BlockSpec(
        (_ROWS_PER_BLOCK, _N // 128, 128), lambda i: (i, 0, 0))
    out_spec = pl.BlockSpec((_ROWS_PER_BLOCK, _N), lambda i: (i, 0))
    c = pl.pallas_call(
        _c_body,
        grid=grid,
        in_specs=[spec, spec],
        out_specs=out_spec,
        out_shape=jax.ShapeDtypeStruct((_B, _N), jnp.float32),
    )(lg3, uu3)
    return (d, c)
